# baseline (speedup 1.0000x reference)
import jax
import jax.numpy as jnp
from jax.experimental import pallas as pl


def _mm_kernel(x_ref, y_ref, o_ref):
    o_ref[0] = jnp.dot(
        x_ref[0],
        y_ref[0].astype(jnp.bfloat16),
        preferred_element_type=jnp.float32,
    )


def kernel(Y, x):
    B, N, _ = Y.shape
    _, C, F, _ = x.shape
    M = C * F
    x2 = x.reshape(B, M, N).astype(jnp.bfloat16)
    out = pl.pallas_call(
        _mm_kernel,
        grid=(B,),
        in_specs=[
            pl.BlockSpec((1, M, N), lambda b: (b, 0, 0)),
            pl.BlockSpec((1, N, N), lambda b: (b, 0, 0)),
        ],
        out_specs=pl.BlockSpec((1, M, N), lambda b: (b, 0, 0)),
        out_shape=jax.ShapeDtypeStruct((B, M, N), jnp.float32),
    )(x2, Y)
    return out.reshape(B, C, F, N)


# speedup vs baseline: 1.0081x; 1.0081x over previous
import jax
import jax.numpy as jnp
from jax.experimental import pallas as pl
from jax.experimental.pallas import tpu as pltpu

_NBUF = 3
_TMC = 256


def _mm_kernel(x_ref, y_hbm, o_ref, buf, acc, sems):
    B, N, _ = y_hbm.shape
    M = x_ref.shape[1]
    K = N // _TMC
    total = B * K

    def copy(i, slot):
        b, k = divmod(i, K)
        return pltpu.make_async_copy(
            y_hbm.at[b, pl.ds(k * _TMC, _TMC), :],
            buf.at[slot],
            sems.at[slot],
        )

    for i in range(_NBUF):
        copy(i, i).start()

    for i in range(total):
        b, k = divmod(i, K)
        slot = i % _NBUF
        copy(i, slot).wait()
        partial = jnp.dot(
            x_ref[b, :, k * _TMC : (k + 1) * _TMC],
            buf[slot].astype(jnp.bfloat16),
            preferred_element_type=jnp.float32,
        )
        if k == 0:
            acc[...] = partial
        else:
            acc[...] += partial
        if k == K - 1:
            o_ref[b] = acc[...]
        if i + _NBUF < total:
            copy(i + _NBUF, slot).start()


def kernel(Y, x):
    B, N, _ = Y.shape
    _, C, F, _ = x.shape
    M = C * F
    x2 = x.reshape(B, M, N).astype(jnp.bfloat16)
    out = pl.pallas_call(
        _mm_kernel,
        in_specs=[
            pl.BlockSpec(memory_space=pltpu.MemorySpace.VMEM),
            pl.BlockSpec(memory_space=pltpu.MemorySpace.HBM),
        ],
        out_specs=pl.BlockSpec(memory_space=pltpu.MemorySpace.VMEM),
        out_shape=jax.ShapeDtypeStruct((B, M, N), jnp.float32),
        scratch_shapes=[
            pltpu.VMEM((_NBUF, _TMC, N), jnp.float32),
            pltpu.VMEM((M, N), jnp.float32),
            pltpu.SemaphoreType.DMA((_NBUF,)),
        ],
    )(x2, Y)
    return out.reshape(B, C, F, N)


# speedup vs baseline: 1.1675x; 1.1581x over previous
import jax
import jax.numpy as jnp
from jax.experimental import pallas as pl
from jax.experimental.pallas import tpu as pltpu


def _mm_kernel(x_ref, y_ref, o_ref):
    o_ref[0] = jnp.dot(
        x_ref[0],
        y_ref[0],
        preferred_element_type=jnp.float32,
    )


def kernel(Y, x):
    B, N, _ = Y.shape
    _, C, F, _ = x.shape
    M = C * F
    x2 = x.reshape(B, M, N).astype(jnp.bfloat16)
    TN = 1024
    out = pl.pallas_call(
        _mm_kernel,
        grid=(B, N // TN),
        in_specs=[
            pl.BlockSpec((1, M, N), lambda b, j: (b, 0, 0)),
            pl.BlockSpec((1, N, TN), lambda b, j: (b, 0, j)),
        ],
        out_specs=pl.BlockSpec((1, M, TN), lambda b, j: (b, 0, j)),
        out_shape=jax.ShapeDtypeStruct((B, M, N), jnp.float32),
        compiler_params=pltpu.CompilerParams(
            allow_input_fusion=[False, True],
        ),
    )(x2, Y.astype(jnp.bfloat16))
    return out.reshape(B, C, F, N)
